# baseline (speedup 1.0000x reference)
import numpy as np
import jax
import jax.numpy as jnp
from jax.experimental import pallas as pl
from jax.experimental.pallas import tpu as pltpu

ROWS = 32
NCOLS = 1000000
BLOCK_W = 8192
NBLK = (NCOLS + BLOCK_W - 1) // BLOCK_W

_TINY = np.float32(np.finfo(np.float32).tiny)
_ONE_MINUS_TINY = np.float32(np.float32(1.0) - _TINY)


def _np_threefry2x32(k1, k2, x1, x2):
    rot = [[13, 15, 26, 6], [17, 29, 16, 24]]

    def rotl(v, r):
        return ((v << np.uint32(r)) | (v >> np.uint32(32 - r))).astype(np.uint32)

    ks = [np.uint32(k1), np.uint32(k2),
          np.uint32(np.uint32(k1) ^ np.uint32(k2) ^ np.uint32(0x1BD11BDA))]
    x1 = (x1 + ks[0]).astype(np.uint32)
    x2 = (x2 + ks[1]).astype(np.uint32)
    for i in range(5):
        for r in rot[i % 2]:
            x1 = (x1 + x2).astype(np.uint32)
            x2 = rotl(x2, r)
            x2 = x2 ^ x1
        x1 = (x1 + ks[(i + 1) % 3]).astype(np.uint32)
        x2 = (x2 + ks[(i + 2) % 3] + np.uint32(i + 1)).astype(np.uint32)
    return x1, x2


_KEY1, _KEY2 = (int(a[0]) for a in _np_threefry2x32(0, 0, np.uint32([0]), np.uint32([1])))


def _gumbel_bits(cnt_u32):
    k1 = jnp.uint32(_KEY1)
    k2 = jnp.uint32(_KEY2)
    k3 = jnp.uint32(_KEY1 ^ _KEY2 ^ 0x1BD11BDA)
    ks = (k1, k2, k3)
    rot = ((13, 15, 26, 6), (17, 29, 16, 24))

    x1 = jnp.full(cnt_u32.shape, k1, dtype=jnp.uint32)
    x2 = cnt_u32 + k2
    for i in range(5):
        for r in rot[i % 2]:
            x1 = x1 + x2
            x2 = (x2 << jnp.uint32(r)) | (x2 >> jnp.uint32(32 - r))
            x2 = x2 ^ x1
        x1 = x1 + ks[(i + 1) % 3]
        x2 = x2 + ks[(i + 2) % 3] + jnp.uint32(i + 1)
    return x1 ^ x2


def _gumbel_noise(col):
    row = jax.lax.broadcasted_iota(jnp.int32, col.shape, 0)
    cnt = (row * NCOLS + col).astype(jnp.uint32)
    bits = _gumbel_bits(cnt)
    u = jax.lax.bitcast_convert_type(
        (bits >> jnp.uint32(9)) | jnp.uint32(0x3F800000), jnp.float32)
    u = u - jnp.float32(1.0)
    u = jnp.maximum(_TINY, u * _ONE_MINUS_TINY + _TINY)
    return -jnp.log(-jnp.log(u))


def _stats_kernel(x_ref, m_ref, s_ref, bi_ref, bv_ref):
    c = pl.program_id(0)

    @pl.when(c == 0)
    def _init():
        m_ref[...] = jnp.full_like(m_ref, -jnp.inf)
        s_ref[...] = jnp.zeros_like(s_ref)
        bi_ref[...] = jnp.zeros_like(bi_ref)
        bv_ref[...] = jnp.full_like(bv_ref, -jnp.inf)

    x = x_ref[...]
    col = jax.lax.broadcasted_iota(jnp.int32, x.shape, 1) + c * BLOCK_W
    valid = col < NCOLS

    xm = jnp.where(valid, x, -jnp.inf)
    m_old = m_ref[...]
    m_new = jnp.maximum(m_old, jnp.max(xm, axis=1, keepdims=True))
    e = jnp.exp(xm - m_new)
    s_ref[...] = s_ref[...] * jnp.exp(m_old - m_new) + jnp.sum(
        e, axis=1, keepdims=True)
    m_ref[...] = m_new

    g = jnp.where(valid, x + _gumbel_noise(col), -jnp.inf)
    gmax = jnp.max(g, axis=1, keepdims=True)
    gidx = jnp.min(jnp.where(g == gmax, col, jnp.int32(0x7FFFFFFF)),
                   axis=1, keepdims=True)
    upd = gmax > bv_ref[...]
    bv_ref[...] = jnp.where(upd, gmax, bv_ref[...])
    bi_ref[...] = jnp.where(upd, gidx, bi_ref[...])


def _emit_kernel(x_ref, m_ref, s_ref, bi_ref, oh_ref, pi_ref):
    c = pl.program_id(0)
    x = x_ref[...]
    inv_s = jnp.float32(1.0) / s_ref[...]
    pi_ref[...] = jnp.exp(x - m_ref[...]) * inv_s
    col = jax.lax.broadcasted_iota(jnp.int32, x.shape, 1) + c * BLOCK_W
    oh_ref[...] = jnp.where(col == bi_ref[...], jnp.float32(1.0),
                            jnp.float32(0.0))


def kernel(logits):
    small = pl.BlockSpec((ROWS, 1), lambda c: (0, 0))
    xspec = pl.BlockSpec((ROWS, BLOCK_W), lambda c: (0, c))

    m, s, bi, _bv = pl.pallas_call(
        _stats_kernel,
        grid=(NBLK,),
        in_specs=[xspec],
        out_specs=[small, small, small, small],
        out_shape=[
            jax.ShapeDtypeStruct((ROWS, 1), jnp.float32),
            jax.ShapeDtypeStruct((ROWS, 1), jnp.float32),
            jax.ShapeDtypeStruct((ROWS, 1), jnp.int32),
            jax.ShapeDtypeStruct((ROWS, 1), jnp.float32),
        ],
        compiler_params=pltpu.CompilerParams(
            dimension_semantics=("arbitrary",)),
    )(logits)

    one_hot, pi = pl.pallas_call(
        _emit_kernel,
        grid=(NBLK,),
        in_specs=[xspec, small, small, small],
        out_specs=[xspec, xspec],
        out_shape=[
            jax.ShapeDtypeStruct((ROWS, NCOLS), jnp.float32),
            jax.ShapeDtypeStruct((ROWS, NCOLS), jnp.float32),
        ],
        compiler_params=pltpu.CompilerParams(
            dimension_semantics=("arbitrary",)),
    )(logits, m, s, bi)

    return (one_hot, pi)
